# Initial kernel scaffold; baseline (speedup 1.0000x reference)
import jax
import jax.numpy as jnp
from jax.experimental import pallas as pl


def kernel(inputs, embeddings):
    raise NotImplementedError("write your pallas kernel here")


# speedup vs baseline: 1.6121x; 1.6121x over previous
import functools

import jax
import jax.numpy as jnp
from jax import lax
from jax.experimental import pallas as pl
from jax.experimental.pallas import tpu as pltpu
from jax.experimental.pallas import tpu_sc as plsc

B, K, D = 8192, 100, 100
D_PAD = 128
BLK = 1024

_NC, _NS = 2, 16
_NW = _NC * _NS
_BPW = B // _NW
_CHUNK = 128
_NCHUNK = _BPW // _CHUNK


def _argmin_body(x_ref, et_ref, idx_ref):
    et = et_ref[...]
    en = jnp.sum(et * et, axis=0, keepdims=True)
    scores = jnp.dot(x_ref[...], et,
                     preferred_element_type=jnp.float32,
                     precision=lax.Precision.HIGHEST)
    total = en - 2.0 * scores
    idx_ref[...] = jnp.argmin(total, axis=1).astype(jnp.int32)


def _tc_argmin(x, et, interpret=False):
    return pl.pallas_call(
        _argmin_body,
        grid=(B // BLK,),
        in_specs=[
            pl.BlockSpec((BLK, D), lambda i: (i, 0)),
            pl.BlockSpec((D, K), lambda i: (0, 0)),
        ],
        out_specs=pl.BlockSpec((BLK,), lambda i: (i,)),
        out_shape=jax.ShapeDtypeStruct((B,), jnp.int32),
        interpret=interpret,
    )(x, et)


def _sc_gather_body(table_hbm, idx_hbm, out_hbm, idx_v, rows_v, sem):
    wid = lax.axis_index("s") * _NC + lax.axis_index("c")
    row0 = wid * _NCHUNK
    pltpu.sync_copy(idx_hbm.at[pl.ds(row0, _NCHUNK)], idx_v)
    copies = [
        pltpu.async_copy(table_hbm.at[idx_v.at[j]], rows_v.at[j], sem)
        for j in range(_NCHUNK)
    ]
    for c in copies:
        c.wait()
    pltpu.sync_copy(rows_v, out_hbm.at[pl.ds(row0, _NCHUNK)])


@functools.cache
def _sc_gather():
    return pl.kernel(
        _sc_gather_body,
        mesh=plsc.VectorSubcoreMesh(core_axis_name="c", subcore_axis_name="s"),
        out_type=jax.ShapeDtypeStruct((_NW * _NCHUNK, _CHUNK, D_PAD),
                                      jnp.float32),
        scratch_types=[
            pltpu.VMEM((_NCHUNK, _CHUNK), jnp.int32),
            pltpu.VMEM((_NCHUNK, _CHUNK, D_PAD), jnp.float32),
            pltpu.SemaphoreType.DMA,
        ],
    )


def kernel(inputs, embeddings):
    idx = _tc_argmin(inputs, embeddings.T)
    ep = jnp.pad(embeddings, ((0, 0), (0, D_PAD - D)))
    out_pad = _sc_gather()(ep, idx.reshape(_NW * _NCHUNK, _CHUNK))
    return out_pad.reshape(B, D_PAD)[:, :D]


# speedup vs baseline: 1.6729x; 1.0377x over previous
import functools

import jax
import jax.numpy as jnp
from jax import lax
from jax.experimental import pallas as pl
from jax.experimental.pallas import tpu as pltpu
from jax.experimental.pallas import tpu_sc as plsc

B, K, D = 8192, 100, 100
D_PAD = 128
BLK = 1024

_NC, _NS = 2, 16
_NW = _NC * _NS
_BPW = B // _NW
_CHUNK = 128
_NCHUNK = _BPW // _CHUNK


def _argmin_body(x_ref, et_ref, idx_ref, ep_ref):
    et = et_ref[...]
    en = jnp.sum(et * et, axis=0, keepdims=True)
    scores = jnp.dot(x_ref[...], et,
                     preferred_element_type=jnp.float32,
                     precision=lax.Precision.HIGHEST)
    total = en - 2.0 * scores
    idx_ref[...] = jnp.argmin(total, axis=1).astype(jnp.int32)

    @pl.when(pl.program_id(0) == 0)
    def _():
        ep_ref[:, :D] = et.T
        ep_ref[:, D:] = jnp.zeros((K, D_PAD - D), jnp.float32)


def _tc_argmin(x, et, interpret=False):
    return pl.pallas_call(
        _argmin_body,
        grid=(B // BLK,),
        in_specs=[
            pl.BlockSpec((BLK, D), lambda i: (i, 0)),
            pl.BlockSpec((D, K), lambda i: (0, 0)),
        ],
        out_specs=[
            pl.BlockSpec((BLK,), lambda i: (i,)),
            pl.BlockSpec((K, D_PAD), lambda i: (0, 0)),
        ],
        out_shape=[
            jax.ShapeDtypeStruct((B,), jnp.int32),
            jax.ShapeDtypeStruct((K, D_PAD), jnp.float32),
        ],
        interpret=interpret,
    )(x, et)


def _sc_gather_body(table_hbm, idx_hbm, out_hbm, idx_v, rows_v, sem):
    wid = lax.axis_index("s") * _NC + lax.axis_index("c")
    row0 = wid * _NCHUNK
    pltpu.sync_copy(idx_hbm.at[pl.ds(row0, _NCHUNK)], idx_v)
    copies = [
        pltpu.async_copy(table_hbm.at[idx_v.at[j]], rows_v.at[j], sem)
        for j in range(_NCHUNK)
    ]
    for c in copies:
        c.wait()
    pltpu.sync_copy(rows_v, out_hbm.at[pl.ds(row0, _NCHUNK)])


@functools.cache
def _sc_gather():
    return pl.kernel(
        _sc_gather_body,
        mesh=plsc.VectorSubcoreMesh(core_axis_name="c", subcore_axis_name="s"),
        out_type=jax.ShapeDtypeStruct((_NW * _NCHUNK, _CHUNK, D_PAD),
                                      jnp.float32),
        scratch_types=[
            pltpu.VMEM((_NCHUNK, _CHUNK), jnp.int32),
            pltpu.VMEM((_NCHUNK, _CHUNK, D_PAD), jnp.float32),
            pltpu.SemaphoreType.DMA,
        ],
    )


def kernel(inputs, embeddings):
    idx, ep = _tc_argmin(inputs, embeddings.T)
    out = _sc_gather()(ep, idx.reshape(_NW * _NCHUNK, _CHUNK))
    return out.reshape(B, D_PAD)[:, :D]


# speedup vs baseline: 1.9235x; 1.1498x over previous
import functools

import jax
import jax.numpy as jnp
from jax import lax
from jax.experimental import pallas as pl
from jax.experimental.pallas import tpu as pltpu
from jax.experimental.pallas import tpu_sc as plsc

B, K, D = 8192, 100, 100
D_PAD = 128
BLK = 1024

_NC, _NS = 2, 16
_NW = _NC * _NS
_BPW = B // _NW
_CHUNK = 128
_NCHUNK = _BPW // _CHUNK


def _argmin_body(x_ref, e_ref, idx_ref, ep_ref):
    e = e_ref[...]
    en = jnp.sum(e * e, axis=1, keepdims=True)
    scores_t = lax.dot_general(e, x_ref[...], (((1,), (1,)), ((), ())),
                               preferred_element_type=jnp.float32,
                               precision=lax.Precision.HIGHEST)
    total = en - 2.0 * scores_t
    idx_ref[...] = jnp.argmin(total, axis=0).astype(jnp.int32)

    @pl.when(pl.program_id(0) == 0)
    def _():
        ep_ref[:, :D] = e
        ep_ref[:, D:] = jnp.zeros((K, D_PAD - D), jnp.float32)


def _tc_argmin(x, e, interpret=False):
    return pl.pallas_call(
        _argmin_body,
        grid=(B // BLK,),
        in_specs=[
            pl.BlockSpec((BLK, D), lambda i: (i, 0)),
            pl.BlockSpec((K, D), lambda i: (0, 0)),
        ],
        out_specs=[
            pl.BlockSpec((BLK,), lambda i: (i,)),
            pl.BlockSpec((K, D_PAD), lambda i: (0, 0)),
        ],
        out_shape=[
            jax.ShapeDtypeStruct((B,), jnp.int32),
            jax.ShapeDtypeStruct((K, D_PAD), jnp.float32),
        ],
        interpret=interpret,
    )(x, e)


def _sc_gather_body(table_hbm, idx_hbm, out_hbm, idx_v, rows_v, sem):
    wid = lax.axis_index("s") * _NC + lax.axis_index("c")
    row0 = wid * _NCHUNK
    pltpu.sync_copy(idx_hbm.at[pl.ds(row0, _NCHUNK)], idx_v)
    copies = [
        pltpu.async_copy(table_hbm.at[idx_v.at[j]], rows_v.at[j], sem)
        for j in range(_NCHUNK)
    ]
    for c in copies:
        c.wait()
    pltpu.sync_copy(rows_v, out_hbm.at[pl.ds(row0, _NCHUNK)])


@functools.cache
def _sc_gather():
    return pl.kernel(
        _sc_gather_body,
        mesh=plsc.VectorSubcoreMesh(core_axis_name="c", subcore_axis_name="s"),
        out_type=jax.ShapeDtypeStruct((_NW * _NCHUNK, _CHUNK, D_PAD),
                                      jnp.float32),
        scratch_types=[
            pltpu.VMEM((_NCHUNK, _CHUNK), jnp.int32),
            pltpu.VMEM((_NCHUNK, _CHUNK, D_PAD), jnp.float32),
            pltpu.SemaphoreType.DMA,
        ],
    )


def kernel(inputs, embeddings):
    idx, ep = _tc_argmin(inputs, embeddings)
    out = _sc_gather()(ep, idx.reshape(_NW * _NCHUNK, _CHUNK))
    return out.reshape(B, D_PAD)[:, :D]
